# baseline (speedup 1.0000x reference)
import functools
import math

import jax
import jax.numpy as jnp
from jax import lax
from jax.experimental import pallas as pl
from jax.experimental.pallas import tpu as pltpu
from jax.experimental.pallas import tpu_sc as plsc

N = 10000
E = 320000
D = 128
D_DOWN = 64
D_EDGE = 16
AVG_NEIGH = 32.0
MLP_IN = D_EDGE + 2 * D_DOWN

NC = 2
NS = 16
NW = NC * NS
CH = 128
NCHUNK = E // CH

SLICES = 5
NCHUNK_S = NCHUNK // SLICES
SLICE_E = NCHUNK_S * CH
SPAN = -(-NCHUNK_S // NW)
IDX_LOAD = 32
NCHUNK_PAD = 2528

_INV_D = 1.0 / math.sqrt(D)
_INV_MLP_IN = 1.0 / math.sqrt(MLP_IN)
_INV_256 = 1.0 / 16.0


def _sc_mesh():
    return plsc.VectorSubcoreMesh(core_axis_name="c", subcore_axis_name="s",
                                  num_cores=NC, num_subcores=NS)


def _worker_span(wid):
    start_l = SPAN * wid
    cnt = jnp.minimum(SPAN, NCHUNK_S - start_l)
    return start_l, cnt


def _load_idx(src_hbm, dst_vmem, gstart):
    off = gstart & 7
    astart = pl.multiple_of(gstart - off, 8)
    pltpu.sync_copy(src_hbm.at[pl.ds(astart, IDX_LOAD)], dst_vmem)
    return off


_ROWS0 = 2048


def _rne16(x):
    bits = jax.lax.bitcast_convert_type(x, jnp.int32)
    return jax.lax.shift_right_logical(
        bits + 0x7FFF + (jax.lax.shift_right_logical(bits, 16) & 1), 16)


def _node_linears_body(nf_ref, wup_ref, wdown_ref, wskip_ref,
                       p_ref, sc_ref):
    nf = nf_ref[...]
    up = jnp.dot(nf, wup_ref[...], preferred_element_type=jnp.float32) * _INV_D
    down = jnp.dot(nf, wdown_ref[...],
                   preferred_element_type=jnp.float32) * _INV_D
    p_ref[...] = _rne16(up) | (_rne16(down) << 16)
    sc_ref[...] = jnp.dot(nf, wskip_ref[...],
                          preferred_element_type=jnp.float32) * _INV_D


def _node_linears(node_feats, W_up, W_down_pad, W_skip):
    return pl.pallas_call(
        _node_linears_body,
        grid=(pl.cdiv(N, _ROWS0),),
        in_specs=[
            pl.BlockSpec((_ROWS0, D), lambda i: (i, 0)),
            pl.BlockSpec((D, D), lambda i: (0, 0)),
            pl.BlockSpec((D, D), lambda i: (0, 0)),
            pl.BlockSpec((D, D), lambda i: (0, 0)),
        ],
        out_specs=[
            pl.BlockSpec((_ROWS0, D), lambda i: (i, 0)),
            pl.BlockSpec((_ROWS0, D), lambda i: (i, 0)),
        ],
        out_shape=[
            jax.ShapeDtypeStruct((N, D), jnp.int32),
            jax.ShapeDtypeStruct((N, D), jnp.float32),
        ],
    )(node_feats, W_up, W_down_pad, W_skip)


def _sc_gather(sender2d, recv2d, P, sl):
    base = sl * NCHUNK_S

    @functools.partial(
        pl.kernel,
        out_type=jax.ShapeDtypeStruct((SLICE_E, D), jnp.int32),
        mesh=_sc_mesh(),
        scratch_types=[
            pltpu.VMEM((IDX_LOAD, CH), jnp.int32),
            pltpu.VMEM((IDX_LOAD, CH), jnp.int32),
            [pltpu.VMEM((CH, D), jnp.int32) for _ in range(3)],
            [pltpu.VMEM((CH, D), jnp.int32) for _ in range(3)],
            [pltpu.SemaphoreType.DMA for _ in range(3)],
            [pltpu.SemaphoreType.DMA for _ in range(3)],
            [pltpu.SemaphoreType.DMA for _ in range(3)],
        ],
        name=f"edge_gather_{sl}",
    )
    def k(sender_hbm, recv_hbm, p_hbm, g_hbm,
          idx_s, idx_r, sbufs, rbufs, ssems, rsems, wsems):
        wid = lax.axis_index("s") * NC + lax.axis_index("c")
        start_l, cnt = _worker_span(wid)

        off = _load_idx(sender_hbm, idx_s, base + start_l)
        _load_idx(recv_hbm, idx_r, base + start_l)

        def fire(kchunk, b):
            pltpu.async_copy(p_hbm.at[idx_s.at[off + kchunk]],
                             sbufs[b], ssems[b])
            pltpu.async_copy(p_hbm.at[idx_r.at[off + kchunk]],
                             rbufs[b], rsems[b])

        def out_slot(kchunk):
            return g_hbm.at[pl.ds(
                pl.multiple_of((start_l + kchunk) * CH, CH), CH)]

        fire(0, 0)
        fire(1, 1)

        def outer(kk, carry):
            for b in range(3):
                kchunk = kk * 3 + b
                bn = (b + 2) % 3

                @pl.when(kchunk < cnt)
                def _(kchunk=kchunk, b=b):
                    pltpu.make_async_copy(p_hbm.at[idx_s.at[off + kchunk]],
                                          sbufs[b], ssems[b]).wait()
                    pltpu.make_async_copy(p_hbm.at[idx_r.at[off + kchunk]],
                                          rbufs[b], rsems[b]).wait()

                    sb, rb = sbufs[b], rbufs[b]

                    def merge(e, c2):
                        for j in range(4):
                            hi = rb[e, pl.ds(j * 16, 16)] & (-65536)
                            sb[e, pl.ds(D_DOWN + j * 16, 16)] = \
                                sb[e, pl.ds(D_DOWN + j * 16, 16)] | hi
                        return c2

                    lax.fori_loop(0, CH, merge, None)
                    pltpu.async_copy(sb, out_slot(kchunk), wsems[b])

                @pl.when((kchunk >= 1) & (kchunk <= cnt))
                def _(kchunk=kchunk, bn=bn):
                    pltpu.make_async_copy(sbufs[bn], out_slot(kchunk - 1),
                                          wsems[bn]).wait()

                @pl.when(kchunk + 2 < cnt)
                def _(kchunk=kchunk, bn=bn):
                    fire(kchunk + 2, bn)

            return carry

        lax.fori_loop(0, -(-(SPAN + 1) // 3), outer, None)

    return k(sender2d, recv2d, P)


_TEDGE = 4000


def _silu(x):
    return (0.5 * x) * (1.0 + jnp.tanh(x * 0.5))


def _edge_mlp_body(ef_ref, g_ref, ea_ref,
                   w1a_ref, w1b_ref, w2_ref, w3_ref, w4_ref, mji_ref):
    gi = g_ref[...]
    ups = jax.lax.bitcast_convert_type(gi << 16, jnp.float32)
    dsdr = jax.lax.bitcast_convert_type(gi & (-65536), jnp.float32)
    h = jnp.dot(ef_ref[...], w1a_ref[...], preferred_element_type=jnp.float32)
    h = h + jnp.dot(dsdr, w1b_ref[...], preferred_element_type=jnp.float32)
    h = _silu(h * _INV_MLP_IN)
    h = _silu(jnp.dot(h, w2_ref[...], preferred_element_type=jnp.float32) * _INV_256)
    h = _silu(jnp.dot(h, w3_ref[...], preferred_element_type=jnp.float32) * _INV_256)
    tpw = jnp.dot(h, w4_ref[...], preferred_element_type=jnp.float32) * _INV_256
    mji_ref[...] = ups * (ea_ref[...] * tpw)


def _edge_mlp(edge_feats, g, edge_attrs, W1a, W1b, W2, W3, W4, sl):
    nblk = SLICE_E // _TEDGE
    off = sl * nblk
    return pl.pallas_call(
        _edge_mlp_body,
        grid=(nblk,),
        in_specs=[
            pl.BlockSpec((_TEDGE, D_EDGE), lambda i: (i + off, 0)),
            pl.BlockSpec((_TEDGE, D), lambda i: (i, 0)),
            pl.BlockSpec((_TEDGE, 1), lambda i: (i + off, 0)),
            pl.BlockSpec((D_EDGE, 256), lambda i: (0, 0)),
            pl.BlockSpec((2 * D_DOWN, 256), lambda i: (0, 0)),
            pl.BlockSpec((256, 256), lambda i: (0, 0)),
            pl.BlockSpec((256, 256), lambda i: (0, 0)),
            pl.BlockSpec((256, D), lambda i: (0, 0)),
        ],
        out_specs=pl.BlockSpec((_TEDGE, D), lambda i: (i, 0)),
        out_shape=jax.ShapeDtypeStruct((SLICE_E, D), jnp.float32),
        name=f"edge_mlp_{sl}",
    )(edge_feats, g, edge_attrs, W1a, W1b, W2, W3, W4)


def _sc_scatter(recv2d, mji, prev, sl):
    base = sl * NCHUNK_S

    @functools.partial(
        pl.kernel,
        out_type=jax.ShapeDtypeStruct((NC, N, D), jnp.float32),
        mesh=_sc_mesh(),
        scratch_types=[
            pltpu.VMEM((IDX_LOAD, CH), jnp.int32),
            [pltpu.VMEM((CH, D), jnp.float32) for _ in range(2)],
            pltpu.VMEM_SHARED((N, D), jnp.float32),
            [pltpu.SemaphoreType.DMA for _ in range(2)],
        ],
        name=f"edge_scatter_{sl}",
    )
    def k(recv_hbm, mji_hbm, prev_hbm, out_hbm,
          idx_r, mbufs, msg_spmem, msems):
        c = lax.axis_index("c")
        s = lax.axis_index("s")
        wid = s * NC + c
        start_l, cnt = _worker_span(wid)

        @pl.when(s == 0)
        def _():
            pltpu.sync_copy(prev_hbm.at[c], msg_spmem)

        off = _load_idx(recv_hbm, idx_r, base + start_l)

        def fire(kchunk, b):
            ebase = pl.multiple_of((start_l + kchunk) * CH, CH)
            pltpu.async_copy(mji_hbm.at[pl.ds(ebase, CH)], mbufs[b], msems[b])

        fire(0, 0)
        fire(1, 1)

        plsc.subcore_barrier()

        def outer(kk, carry):
            for b in range(2):
                kchunk = kk * 2 + b

                @pl.when(kchunk < cnt)
                def _(kchunk=kchunk, b=b):
                    ebase = pl.multiple_of((start_l + kchunk) * CH, CH)
                    pltpu.make_async_copy(mji_hbm.at[pl.ds(ebase, CH)],
                                          mbufs[b], msems[b]).wait()
                    pltpu.sync_copy(mbufs[b],
                                    msg_spmem.at[idx_r.at[off + kchunk]],
                                    add=True)

                    @pl.when(kchunk + 2 < cnt)
                    def _():
                        fire(kchunk + 2, b)

            return carry

        lax.fori_loop(0, SPAN // 2, outer, None)

        plsc.subcore_barrier()

        @pl.when(s == 0)
        def _():
            pltpu.sync_copy(msg_spmem, out_hbm.at[c])

    return k(recv2d, mji, prev)


_ROWSD = 2048


def _finalize_body(p_ref, wlin_ref, out_ref):
    m = p_ref[0] + p_ref[1]
    out_ref[...] = jnp.dot(m, wlin_ref[...], preferred_element_type=jnp.float32) \
        * (_INV_D / AVG_NEIGH)


def _finalize(partials, W_lin):
    return pl.pallas_call(
        _finalize_body,
        grid=(pl.cdiv(N, _ROWSD),),
        in_specs=[
            pl.BlockSpec((NC, _ROWSD, D), lambda i: (0, i, 0)),
            pl.BlockSpec((D, D), lambda i: (0, 0)),
        ],
        out_specs=pl.BlockSpec((_ROWSD, D), lambda i: (i, 0)),
        out_shape=jax.ShapeDtypeStruct((N, D), jnp.float32),
    )(partials, W_lin)


def kernel(node_attrs, node_feats, edge_attrs, edge_feats, edge_index,
           W_up, W_down, W1, W2, W3, W4, W_lin, W_skip):
    sender2d = jnp.pad(edge_index[0].reshape(NCHUNK, CH),
                       ((0, NCHUNK_PAD - NCHUNK), (0, 0)))
    recv2d = jnp.pad(edge_index[1].reshape(NCHUNK, CH),
                     ((0, NCHUNK_PAD - NCHUNK), (0, 0)))
    W_down_pad = jnp.pad(W_down, ((0, 0), (0, D - D_DOWN)))
    P, sc = _node_linears(node_feats, W_up, W_down_pad, W_skip)
    W1a, W1b = W1[:D_EDGE], W1[D_EDGE:]

    gs = [_sc_gather(sender2d, recv2d, P, sl) for sl in range(SLICES)]
    mjis = [_edge_mlp(edge_feats, gs[sl], edge_attrs,
                      W1a, W1b, W2, W3, W4, sl) for sl in range(SLICES)]
    partials = jnp.zeros((NC, N, D), jnp.float32)
    for sl in range(SLICES):
        partials = _sc_scatter(recv2d, mjis[sl], partials, sl)

    message = _finalize(partials, W_lin)
    return message.reshape(N, D, 1), sc


# speedup vs baseline: 1.0135x; 1.0135x over previous
import functools
import math

import jax
import jax.numpy as jnp
from jax import lax
from jax.experimental import pallas as pl
from jax.experimental.pallas import tpu as pltpu
from jax.experimental.pallas import tpu_sc as plsc

N = 10000
E = 320000
D = 128
D_DOWN = 64
D_EDGE = 16
AVG_NEIGH = 32.0
MLP_IN = D_EDGE + 2 * D_DOWN

NC = 2
NS = 16
NW = NC * NS
CH = 128
NCHUNK = E // CH

SLICES = 4
NCHUNK_S = NCHUNK // SLICES
SLICE_E = NCHUNK_S * CH
SPAN = -(-NCHUNK_S // NW)
IDX_LOAD = 32
NCHUNK_PAD = 2528

_INV_D = 1.0 / math.sqrt(D)
_INV_MLP_IN = 1.0 / math.sqrt(MLP_IN)
_INV_256 = 1.0 / 16.0


def _sc_mesh():
    return plsc.VectorSubcoreMesh(core_axis_name="c", subcore_axis_name="s",
                                  num_cores=NC, num_subcores=NS)


def _worker_span(wid):
    start_l = SPAN * wid
    cnt = jnp.minimum(SPAN, NCHUNK_S - start_l)
    return start_l, cnt


def _load_idx(src_hbm, dst_vmem, gstart):
    off = gstart & 7
    astart = pl.multiple_of(gstart - off, 8)
    pltpu.sync_copy(src_hbm.at[pl.ds(astart, IDX_LOAD)], dst_vmem)
    return off


_ROWS0 = 2048


def _rne16(x):
    bits = jax.lax.bitcast_convert_type(x, jnp.int32)
    return jax.lax.shift_right_logical(
        bits + 0x7FFF + (jax.lax.shift_right_logical(bits, 16) & 1), 16)


def _node_linears_body(nf_ref, wup_ref, wdown_ref, wskip_ref,
                       p_ref, sc_ref):
    nf = nf_ref[...]
    up = jnp.dot(nf, wup_ref[...], preferred_element_type=jnp.float32) * _INV_D
    down = jnp.dot(nf, wdown_ref[...],
                   preferred_element_type=jnp.float32) * _INV_D
    p_ref[...] = _rne16(up) | (_rne16(down) << 16)
    sc_ref[...] = jnp.dot(nf, wskip_ref[...],
                          preferred_element_type=jnp.float32) * _INV_D


def _node_linears(node_feats, W_up, W_down_pad, W_skip):
    return pl.pallas_call(
        _node_linears_body,
        grid=(pl.cdiv(N, _ROWS0),),
        in_specs=[
            pl.BlockSpec((_ROWS0, D), lambda i: (i, 0)),
            pl.BlockSpec((D, D), lambda i: (0, 0)),
            pl.BlockSpec((D, D), lambda i: (0, 0)),
            pl.BlockSpec((D, D), lambda i: (0, 0)),
        ],
        out_specs=[
            pl.BlockSpec((_ROWS0, D), lambda i: (i, 0)),
            pl.BlockSpec((_ROWS0, D), lambda i: (i, 0)),
        ],
        out_shape=[
            jax.ShapeDtypeStruct((N, D), jnp.int32),
            jax.ShapeDtypeStruct((N, D), jnp.float32),
        ],
    )(node_feats, W_up, W_down_pad, W_skip)


def _sc_gather(sender2d, recv2d, P, sl):
    base = sl * NCHUNK_S

    @functools.partial(
        pl.kernel,
        out_type=jax.ShapeDtypeStruct((SLICE_E, D), jnp.int32),
        mesh=_sc_mesh(),
        scratch_types=[
            pltpu.VMEM((IDX_LOAD, CH), jnp.int32),
            pltpu.VMEM((IDX_LOAD, CH), jnp.int32),
            [pltpu.VMEM((CH, D), jnp.int32) for _ in range(3)],
            [pltpu.VMEM((CH, D), jnp.int32) for _ in range(3)],
            [pltpu.SemaphoreType.DMA for _ in range(3)],
            [pltpu.SemaphoreType.DMA for _ in range(3)],
            [pltpu.SemaphoreType.DMA for _ in range(3)],
        ],
        name=f"edge_gather_{sl}",
    )
    def k(sender_hbm, recv_hbm, p_hbm, g_hbm,
          idx_s, idx_r, sbufs, rbufs, ssems, rsems, wsems):
        wid = lax.axis_index("s") * NC + lax.axis_index("c")
        start_l, cnt = _worker_span(wid)

        off = _load_idx(sender_hbm, idx_s, base + start_l)
        _load_idx(recv_hbm, idx_r, base + start_l)

        def fire(kchunk, b):
            pltpu.async_copy(p_hbm.at[idx_s.at[off + kchunk]],
                             sbufs[b], ssems[b])
            pltpu.async_copy(p_hbm.at[idx_r.at[off + kchunk]],
                             rbufs[b], rsems[b])

        def out_slot(kchunk):
            return g_hbm.at[pl.ds(
                pl.multiple_of((start_l + kchunk) * CH, CH), CH)]

        fire(0, 0)
        fire(1, 1)

        def outer(kk, carry):
            for b in range(3):
                kchunk = kk * 3 + b
                bn = (b + 2) % 3

                @pl.when(kchunk < cnt)
                def _(kchunk=kchunk, b=b):
                    pltpu.make_async_copy(p_hbm.at[idx_s.at[off + kchunk]],
                                          sbufs[b], ssems[b]).wait()
                    pltpu.make_async_copy(p_hbm.at[idx_r.at[off + kchunk]],
                                          rbufs[b], rsems[b]).wait()

                    sb, rb = sbufs[b], rbufs[b]

                    def merge(e, c2):
                        for j in range(4):
                            hi = rb[e, pl.ds(j * 16, 16)] & (-65536)
                            sb[e, pl.ds(D_DOWN + j * 16, 16)] = \
                                sb[e, pl.ds(D_DOWN + j * 16, 16)] | hi
                        return c2

                    lax.fori_loop(0, CH, merge, None)
                    pltpu.async_copy(sb, out_slot(kchunk), wsems[b])

                @pl.when((kchunk >= 1) & (kchunk <= cnt))
                def _(kchunk=kchunk, bn=bn):
                    pltpu.make_async_copy(sbufs[bn], out_slot(kchunk - 1),
                                          wsems[bn]).wait()

                @pl.when(kchunk + 2 < cnt)
                def _(kchunk=kchunk, bn=bn):
                    fire(kchunk + 2, bn)

            return carry

        lax.fori_loop(0, -(-(SPAN + 1) // 3), outer, None)

    return k(sender2d, recv2d, P)


_TEDGE = 8000


def _silu(x):
    return (0.5 * x) * (1.0 + jnp.tanh(x * 0.5))


def _edge_mlp_body(ef_ref, g_ref, ea_ref,
                   w1a_ref, w1b_ref, w2_ref, w3_ref, w4_ref, mji_ref):
    gi = g_ref[...]
    ups = jax.lax.bitcast_convert_type(gi << 16, jnp.float32)
    dsdr = jax.lax.bitcast_convert_type(gi & (-65536), jnp.float32)
    h = jnp.dot(ef_ref[...], w1a_ref[...], preferred_element_type=jnp.float32)
    h = h + jnp.dot(dsdr, w1b_ref[...], preferred_element_type=jnp.float32)
    h = _silu(h * _INV_MLP_IN)
    h = _silu(jnp.dot(h, w2_ref[...], preferred_element_type=jnp.float32) * _INV_256)
    h = _silu(jnp.dot(h, w3_ref[...], preferred_element_type=jnp.float32) * _INV_256)
    tpw = jnp.dot(h, w4_ref[...], preferred_element_type=jnp.float32) * _INV_256
    mji_ref[...] = ups * (ea_ref[...] * tpw)


def _edge_mlp(edge_feats, g, edge_attrs, W1a, W1b, W2, W3, W4, sl):
    nblk = SLICE_E // _TEDGE
    off = sl * nblk
    return pl.pallas_call(
        _edge_mlp_body,
        grid=(nblk,),
        in_specs=[
            pl.BlockSpec((_TEDGE, D_EDGE), lambda i: (i + off, 0)),
            pl.BlockSpec((_TEDGE, D), lambda i: (i, 0)),
            pl.BlockSpec((_TEDGE, 1), lambda i: (i + off, 0)),
            pl.BlockSpec((D_EDGE, 256), lambda i: (0, 0)),
            pl.BlockSpec((2 * D_DOWN, 256), lambda i: (0, 0)),
            pl.BlockSpec((256, 256), lambda i: (0, 0)),
            pl.BlockSpec((256, 256), lambda i: (0, 0)),
            pl.BlockSpec((256, D), lambda i: (0, 0)),
        ],
        out_specs=pl.BlockSpec((_TEDGE, D), lambda i: (i, 0)),
        out_shape=jax.ShapeDtypeStruct((SLICE_E, D), jnp.float32),
        name=f"edge_mlp_{sl}",
    )(edge_feats, g, edge_attrs, W1a, W1b, W2, W3, W4)


def _sc_scatter(recv2d, mji, prev, sl):
    base = sl * NCHUNK_S

    @functools.partial(
        pl.kernel,
        out_type=jax.ShapeDtypeStruct((NC, N, D), jnp.float32),
        mesh=_sc_mesh(),
        scratch_types=[
            pltpu.VMEM((IDX_LOAD, CH), jnp.int32),
            [pltpu.VMEM((CH, D), jnp.float32) for _ in range(2)],
            pltpu.VMEM_SHARED((N, D), jnp.float32),
            [pltpu.SemaphoreType.DMA for _ in range(2)],
        ],
        name=f"edge_scatter_{sl}",
    )
    def k(recv_hbm, mji_hbm, prev_hbm, out_hbm,
          idx_r, mbufs, msg_spmem, msems):
        c = lax.axis_index("c")
        s = lax.axis_index("s")
        wid = s * NC + c
        start_l, cnt = _worker_span(wid)

        @pl.when(s == 0)
        def _():
            pltpu.sync_copy(prev_hbm.at[c], msg_spmem)

        off = _load_idx(recv_hbm, idx_r, base + start_l)

        def fire(kchunk, b):
            ebase = pl.multiple_of((start_l + kchunk) * CH, CH)
            pltpu.async_copy(mji_hbm.at[pl.ds(ebase, CH)], mbufs[b], msems[b])

        fire(0, 0)
        fire(1, 1)

        plsc.subcore_barrier()

        def outer(kk, carry):
            for b in range(2):
                kchunk = kk * 2 + b

                @pl.when(kchunk < cnt)
                def _(kchunk=kchunk, b=b):
                    ebase = pl.multiple_of((start_l + kchunk) * CH, CH)
                    pltpu.make_async_copy(mji_hbm.at[pl.ds(ebase, CH)],
                                          mbufs[b], msems[b]).wait()
                    pltpu.sync_copy(mbufs[b],
                                    msg_spmem.at[idx_r.at[off + kchunk]],
                                    add=True)

                    @pl.when(kchunk + 2 < cnt)
                    def _():
                        fire(kchunk + 2, b)

            return carry

        lax.fori_loop(0, SPAN // 2, outer, None)

        plsc.subcore_barrier()

        @pl.when(s == 0)
        def _():
            pltpu.sync_copy(msg_spmem, out_hbm.at[c])

    return k(recv2d, mji, prev)


_ROWSD = 2048


def _finalize_body(p_ref, wlin_ref, out_ref):
    m = p_ref[0] + p_ref[1]
    out_ref[...] = jnp.dot(m, wlin_ref[...], preferred_element_type=jnp.float32) \
        * (_INV_D / AVG_NEIGH)


def _finalize(partials, W_lin):
    return pl.pallas_call(
        _finalize_body,
        grid=(pl.cdiv(N, _ROWSD),),
        in_specs=[
            pl.BlockSpec((NC, _ROWSD, D), lambda i: (0, i, 0)),
            pl.BlockSpec((D, D), lambda i: (0, 0)),
        ],
        out_specs=pl.BlockSpec((_ROWSD, D), lambda i: (i, 0)),
        out_shape=jax.ShapeDtypeStruct((N, D), jnp.float32),
    )(partials, W_lin)


def kernel(node_attrs, node_feats, edge_attrs, edge_feats, edge_index,
           W_up, W_down, W1, W2, W3, W4, W_lin, W_skip):
    sender2d = jnp.pad(edge_index[0].reshape(NCHUNK, CH),
                       ((0, NCHUNK_PAD - NCHUNK), (0, 0)))
    recv2d = jnp.pad(edge_index[1].reshape(NCHUNK, CH),
                     ((0, NCHUNK_PAD - NCHUNK), (0, 0)))
    W_down_pad = jnp.pad(W_down, ((0, 0), (0, D - D_DOWN)))
    P, sc = _node_linears(node_feats, W_up, W_down_pad, W_skip)
    W1a, W1b = W1[:D_EDGE], W1[D_EDGE:]

    gs = [_sc_gather(sender2d, recv2d, P, sl) for sl in range(SLICES)]
    mjis = [_edge_mlp(edge_feats, gs[sl], edge_attrs,
                      W1a, W1b, W2, W3, W4, sl) for sl in range(SLICES)]
    partials = jnp.zeros((NC, N, D), jnp.float32)
    for sl in range(SLICES):
        partials = _sc_scatter(recv2d, mjis[sl], partials, sl)

    message = _finalize(partials, W_lin)
    return message.reshape(N, D, 1), sc
